# baseline (speedup 1.0000x reference)
import functools

import jax
import jax.numpy as jnp
from jax import lax
from jax.experimental import pallas as pl
from jax.experimental.pallas import tpu as pltpu
from jax.experimental.pallas import tpu_sc as plsc

B = 4096
NH = 2
BH = B // NH
PATH_LEN = 50
CHAR_LEN = 200
D = 64
SEQ = 1 + PATH_LEN + 1 + CHAR_LEN
EPS = 1e-5

NC, NS = 2, 16
GROWS = 128
GPC = 4
CHUNK = GROWS * GPC
N_REAL = BH * (CHAR_LEN + 2)
CPW_FAST = 32
CPW_SLOW = 20
NPAD = NS * (CPW_FAST + CPW_SLOW) * CHUNK
CS_PAIR0 = BH * CHAR_LEN // 2


def _sc_gather_body(idx_hbm, table_hbm, out_hbm, idx_v, rows_v,
                    sidx0, sidx1, sg0, sg1, so0, so1):
    core = lax.axis_index("c")
    sub = lax.axis_index("s")
    base_chunk = jnp.where(core == 0, sub * CPW_FAST,
                           NS * CPW_FAST + sub * CPW_SLOW)
    count = jnp.where(core == 0, CPW_FAST, CPW_SLOW)
    sidx = (sidx0, sidx1)
    sg = (sg0, sg1)
    so = (so0, so1)

    def idx_start(c, b):
        pltpu.async_copy(idx_hbm.at[pl.ds((base_chunk + c) * GPC, GPC)],
                         idx_v.at[b], sidx[b])

    idx_start(0, 0)
    idx_start(1, 1)

    def it_body(it, carry):
        for b in range(2):
            c = 2 * it + b

            @pl.when(it > 0)
            def _():
                pltpu.make_async_copy(
                    rows_v.at[b], out_hbm.at[pl.ds(0, CHUNK)], so[b]).wait()

            pltpu.make_async_copy(
                idx_hbm.at[pl.ds(0, GPC)], idx_v.at[b], sidx[b]).wait()

            gathers = [
                pltpu.async_copy(
                    table_hbm.at[idx_v.at[b].at[j]],
                    rows_v.at[b].at[pl.ds(j * GROWS, GROWS)],
                    sg[b],
                )
                for j in range(GPC)
            ]
            for cp in gathers:
                cp.wait()

            @pl.when(c + 2 < count)
            def _():
                idx_start(c + 2, b)

            pltpu.async_copy(
                rows_v.at[b],
                out_hbm.at[pl.ds((base_chunk + c) * CHUNK, CHUNK)],
                so[b])
        return carry

    lax.fori_loop(0, count // 2, it_body, 0)
    for b in range(2):
        pltpu.make_async_copy(
            rows_v.at[b], out_hbm.at[pl.ds(0, CHUNK)], so[b]).wait()


@functools.lru_cache(maxsize=1)
def _sc_gather():
    return pl.kernel(
        _sc_gather_body,
        mesh=plsc.VectorSubcoreMesh(core_axis_name="c", subcore_axis_name="s"),
        out_type=jax.ShapeDtypeStruct((NPAD, D), jnp.float32),
        scratch_types=[
            pltpu.VMEM((2, GPC, GROWS), jnp.int32),
            pltpu.VMEM((2, CHUNK, D), jnp.float32),
            pltpu.SemaphoreType.DMA,
            pltpu.SemaphoreType.DMA,
            pltpu.SemaphoreType.DMA,
            pltpu.SemaphoreType.DMA,
            pltpu.SemaphoreType.DMA,
            pltpu.SemaphoreType.DMA,
        ],
        compiler_params=pltpu.CompilerParams(use_tc_tiling_on_sc=False),
    )


NB = 64
NPAIR_LO = 26
NPAIR_HI = 100
NPAIRS = NPAIR_LO + NPAIR_HI


def _tc_finish(char_ref, cs_ref, cevx_ref, cevy_ref, cevz_ref,
               codx_ref, cody_ref, codz_ref, w_ref, b_ref,
               pospair_ref, type_ref, gamma_ref, beta_ref, out_ref):
    g2 = jnp.concatenate([gamma_ref[...], gamma_ref[...]], axis=-1)
    b2 = jnp.concatenate([beta_ref[...], beta_ref[...]], axis=-1)
    t0 = type_ref[0:1, :]
    t00 = jnp.concatenate([t0, t0], axis=-1)[None]
    t1 = type_ref[1:2, :]
    t11 = jnp.concatenate([t1, t1], axis=-1)[None]

    ri = lax.broadcasted_iota(jnp.int32, (128, 128), 0) // 64
    ci = lax.broadcasted_iota(jnp.int32, (128, 128), 1) // 64
    m_half = (ri == ci).astype(jnp.float32)

    def ln2(x):
        r, p, _ = x.shape
        x2 = x.reshape(r * p, 128)
        s = lax.dot(x2, m_half)
        q = lax.dot(x2 * x2, m_half)
        mu = s * (1.0 / 64.0)
        v = q * (1.0 / 64.0) - mu * mu
        y = (x2 - mu) * lax.rsqrt(v + EPS) * g2 + b2
        return y.reshape(r, p, 128)

    w0 = w_ref[0:1, :]
    w1 = w_ref[1:2, :]
    w2 = w_ref[2:3, :]
    bp = b_ref[...]
    pe_l = (cevx_ref[...][:, :, None] * w0 + cevy_ref[...][:, :, None] * w1
            + cevz_ref[...][:, :, None] * w2 + bp)
    pe_r = (codx_ref[...][:, :, None] * w0 + cody_ref[...][:, :, None] * w1
            + codz_ref[...][:, :, None] * w2 + bp)
    lo = jnp.concatenate([pe_l, pe_r], axis=-1)
    pi = lax.broadcasted_iota(jnp.int32, (1, NPAIR_LO, 128), 1)
    li = lax.broadcasted_iota(jnp.int32, (1, NPAIR_LO, 128), 2)
    cs_mask = ((pi == 0) & (li < D)) | ((pi == NPAIR_LO - 1) & (li >= D))
    lo = jnp.where(cs_mask, cs_ref[...][:, None, :], lo)
    x_lo = lo + pospair_ref[0:NPAIR_LO, :][None] + t00
    out_ref[:, 0:NPAIR_LO, :] = ln2(x_lo)

    x_hi = (char_ref[...].reshape(NB, NPAIR_HI, 2 * D)
            + pospair_ref[NPAIR_LO:NPAIRS, :][None] + t11)
    out_ref[:, NPAIR_LO:NPAIRS, :] = ln2(x_hi)


def _build_idx(char_tokens, cls_token, sep_token):
    return jnp.concatenate([
        char_tokens.reshape(-1).astype(jnp.int32),
        jnp.concatenate([cls_token.astype(jnp.int32),
                         sep_token.astype(jnp.int32)], axis=1).reshape(-1),
        jnp.zeros((NPAD - N_REAL,), jnp.int32),
    ])


def _tc_call(staging_pairs, path_coords, W_path, b_path, pos_table,
             type_table, gamma, beta, interpret=False):
    zero3 = jnp.zeros((BH, 1, 3), path_coords.dtype)
    cext = jnp.concatenate([zero3, path_coords, zero3], axis=1)
    cev = cext[:, 0::2, :]
    cod = cext[:, 1::2, :]

    grid = (BH // NB,)
    cs_block0 = CS_PAIR0 // NB
    out = pl.pallas_call(
        _tc_finish,
        grid=grid,
        in_specs=[
            pl.BlockSpec((NB * NPAIR_HI, 2 * D), lambda i: (i, 0)),
            pl.BlockSpec((NB, 2 * D), lambda i: (cs_block0 + i, 0)),
            pl.BlockSpec((NB, NPAIR_LO), lambda i: (i, 0)),
            pl.BlockSpec((NB, NPAIR_LO), lambda i: (i, 0)),
            pl.BlockSpec((NB, NPAIR_LO), lambda i: (i, 0)),
            pl.BlockSpec((NB, NPAIR_LO), lambda i: (i, 0)),
            pl.BlockSpec((NB, NPAIR_LO), lambda i: (i, 0)),
            pl.BlockSpec((NB, NPAIR_LO), lambda i: (i, 0)),
            pl.BlockSpec((3, D), lambda i: (0, 0)),
            pl.BlockSpec((1, D), lambda i: (0, 0)),
            pl.BlockSpec((NPAIRS, 2 * D), lambda i: (0, 0)),
            pl.BlockSpec((2, D), lambda i: (0, 0)),
            pl.BlockSpec((1, D), lambda i: (0, 0)),
            pl.BlockSpec((1, D), lambda i: (0, 0)),
        ],
        out_specs=pl.BlockSpec((NB, NPAIRS, 2 * D), lambda i: (i, 0, 0)),
        out_shape=jax.ShapeDtypeStruct((BH, NPAIRS, 2 * D), jnp.float32),
        interpret=interpret,
    )(staging_pairs, staging_pairs,
      cev[:, :, 0], cev[:, :, 1], cev[:, :, 2],
      cod[:, :, 0], cod[:, :, 1], cod[:, :, 2],
      W_path, b_path.reshape(1, D), pos_table.reshape(NPAIRS, 2 * D),
      type_table, gamma.reshape(1, D), beta.reshape(1, D))
    return out


def kernel(path_coords, char_tokens, cls_token, sep_token, W_path, b_path,
           char_table, pos_table, type_table, gamma, beta):
    outs = []
    for h in range(NH):
        lo, hi = h * BH, (h + 1) * BH
        idx = _build_idx(char_tokens[lo:hi], cls_token[lo:hi],
                         sep_token[lo:hi]).reshape(NPAD // GROWS, GROWS)
        staging = _sc_gather()(idx, char_table)
        staging_pairs = staging.reshape(NPAD // 2, 2 * D)
        outs.append(_tc_call(staging_pairs, path_coords[lo:hi], W_path,
                             b_path, pos_table, type_table, gamma, beta))
    return jnp.concatenate(outs, axis=0).reshape(B, SEQ, D)


# speedup vs baseline: 1.2962x; 1.2962x over previous
import functools

import jax
import jax.numpy as jnp
from jax import lax
from jax.experimental import pallas as pl
from jax.experimental.pallas import tpu as pltpu
from jax.experimental.pallas import tpu_sc as plsc

B = 4096
PATH_LEN = 50
CHAR_LEN = 200
D = 64
SEQ = 1 + PATH_LEN + 1 + CHAR_LEN
EPS = 1e-5

NC, NS = 2, 16
NW = NC * NS
GROWS = 128
GPC = 4
CHUNK = GROWS * GPC
N_REAL = B * (CHAR_LEN + 2)
CPW = -(-N_REAL // (NW * CHUNK))
NPAD = NW * CPW * CHUNK
CPW_FAST = 80
CPW_SLOW = 2 * CPW - CPW_FAST
CS_PAIR0 = B * CHAR_LEN // 2


def _sc_gather_body(idx_hbm, table_hbm, out_hbm, idx_v, rows_v,
                    sidx0, sidx1, sg0, sg1, so0, so1):
    core = lax.axis_index("c")
    sub = lax.axis_index("s")
    base_chunk = jnp.where(core == 0, sub * CPW_FAST,
                           NS * CPW_FAST + sub * CPW_SLOW)
    count = jnp.where(core == 0, CPW_FAST, CPW_SLOW)
    sidx = (sidx0, sidx1)
    sg = (sg0, sg1)
    so = (so0, so1)

    def idx_start(c, b):
        pltpu.async_copy(idx_hbm.at[pl.ds((base_chunk + c) * GPC, GPC)],
                         idx_v.at[b], sidx[b])

    idx_start(0, 0)
    idx_start(1, 1)

    def it_body(it, carry):
        for b in range(2):
            c = 2 * it + b

            @pl.when(it > 0)
            def _():
                pltpu.make_async_copy(
                    rows_v.at[b], out_hbm.at[pl.ds(0, CHUNK)], so[b]).wait()

            pltpu.make_async_copy(
                idx_hbm.at[pl.ds(0, GPC)], idx_v.at[b], sidx[b]).wait()

            gathers = [
                pltpu.async_copy(
                    table_hbm.at[idx_v.at[b].at[j]],
                    rows_v.at[b].at[pl.ds(j * GROWS, GROWS)],
                    sg[b],
                )
                for j in range(GPC)
            ]
            for cp in gathers:
                cp.wait()

            @pl.when(c + 2 < count)
            def _():
                idx_start(c + 2, b)

            pltpu.async_copy(
                rows_v.at[b],
                out_hbm.at[pl.ds((base_chunk + c) * CHUNK, CHUNK)],
                so[b])
        return carry

    lax.fori_loop(0, count // 2, it_body, 0)
    for b in range(2):
        pltpu.make_async_copy(
            rows_v.at[b], out_hbm.at[pl.ds(0, CHUNK)], so[b]).wait()


@functools.lru_cache(maxsize=1)
def _sc_gather():
    return pl.kernel(
        _sc_gather_body,
        mesh=plsc.VectorSubcoreMesh(core_axis_name="c", subcore_axis_name="s"),
        out_type=jax.ShapeDtypeStruct((NPAD, D), jnp.float32),
        scratch_types=[
            pltpu.VMEM((2, GPC, GROWS), jnp.int32),
            pltpu.VMEM((2, CHUNK, D), jnp.float32),
            pltpu.SemaphoreType.DMA,
            pltpu.SemaphoreType.DMA,
            pltpu.SemaphoreType.DMA,
            pltpu.SemaphoreType.DMA,
            pltpu.SemaphoreType.DMA,
            pltpu.SemaphoreType.DMA,
        ],
        compiler_params=pltpu.CompilerParams(use_tc_tiling_on_sc=False),
    )


NB = 64
NPAIR_LO = 26
NPAIR_HI = 100
NPAIRS = NPAIR_LO + NPAIR_HI


def _tc_finish(char_ref, cs_ref, cevx_ref, cevy_ref, cevz_ref,
               codx_ref, cody_ref, codz_ref, w_ref, b_ref,
               pospair_ref, type_ref, gamma_ref, beta_ref, out_ref):
    g2 = jnp.concatenate([gamma_ref[...], gamma_ref[...]], axis=-1)
    b2 = jnp.concatenate([beta_ref[...], beta_ref[...]], axis=-1)
    t0 = type_ref[0:1, :]
    t00 = jnp.concatenate([t0, t0], axis=-1)[None]
    t1 = type_ref[1:2, :]
    t11 = jnp.concatenate([t1, t1], axis=-1)[None]

    ri = lax.broadcasted_iota(jnp.int32, (128, 128), 0) // 64
    ci = lax.broadcasted_iota(jnp.int32, (128, 128), 1) // 64
    m_half = (ri == ci).astype(jnp.float32)

    def ln2(x):
        r, p, _ = x.shape
        x2 = x.reshape(r * p, 128)
        s = lax.dot(x2, m_half)
        q = lax.dot(x2 * x2, m_half)
        mu = s * (1.0 / 64.0)
        v = q * (1.0 / 64.0) - mu * mu
        y = (x2 - mu) * lax.rsqrt(v + EPS) * g2 + b2
        return y.reshape(r, p, 128)

    w0 = w_ref[0:1, :]
    w1 = w_ref[1:2, :]
    w2 = w_ref[2:3, :]
    bp = b_ref[...]
    pe_l = (cevx_ref[...][:, :, None] * w0 + cevy_ref[...][:, :, None] * w1
            + cevz_ref[...][:, :, None] * w2 + bp)
    pe_r = (codx_ref[...][:, :, None] * w0 + cody_ref[...][:, :, None] * w1
            + codz_ref[...][:, :, None] * w2 + bp)
    lo = jnp.concatenate([pe_l, pe_r], axis=-1)
    pi = lax.broadcasted_iota(jnp.int32, (1, NPAIR_LO, 128), 1)
    li = lax.broadcasted_iota(jnp.int32, (1, NPAIR_LO, 128), 2)
    cs_mask = ((pi == 0) & (li < D)) | ((pi == NPAIR_LO - 1) & (li >= D))
    lo = jnp.where(cs_mask, cs_ref[...][:, None, :], lo)
    x_lo = lo + pospair_ref[0:NPAIR_LO, :][None] + t00
    out_ref[:, 0:NPAIR_LO, :] = ln2(x_lo)

    x_hi = (char_ref[...].reshape(NB, NPAIR_HI, 2 * D)
            + pospair_ref[NPAIR_LO:NPAIRS, :][None] + t11)
    out_ref[:, NPAIR_LO:NPAIRS, :] = ln2(x_hi)


def _build_idx(char_tokens, cls_token, sep_token):
    return jnp.concatenate([
        char_tokens.reshape(-1).astype(jnp.int32),
        jnp.concatenate([cls_token.astype(jnp.int32),
                         sep_token.astype(jnp.int32)], axis=1).reshape(-1),
        jnp.zeros((NPAD - N_REAL,), jnp.int32),
    ])


def _tc_call(staging_pairs, path_coords, W_path, b_path, pos_table,
             type_table, gamma, beta, interpret=False):

    zero3 = jnp.zeros((B, 1, 3), path_coords.dtype)
    cext = jnp.concatenate([zero3, path_coords, zero3], axis=1)
    cev = cext[:, 0::2, :]
    cod = cext[:, 1::2, :]

    grid = (B // NB,)
    cs_block0 = CS_PAIR0 // NB
    out = pl.pallas_call(
        _tc_finish,
        grid=grid,
        in_specs=[
            pl.BlockSpec((NB * NPAIR_HI, 2 * D), lambda i: (i, 0)),
            pl.BlockSpec((NB, 2 * D), lambda i: (cs_block0 + i, 0)),
            pl.BlockSpec((NB, NPAIR_LO), lambda i: (i, 0)),
            pl.BlockSpec((NB, NPAIR_LO), lambda i: (i, 0)),
            pl.BlockSpec((NB, NPAIR_LO), lambda i: (i, 0)),
            pl.BlockSpec((NB, NPAIR_LO), lambda i: (i, 0)),
            pl.BlockSpec((NB, NPAIR_LO), lambda i: (i, 0)),
            pl.BlockSpec((NB, NPAIR_LO), lambda i: (i, 0)),
            pl.BlockSpec((3, D), lambda i: (0, 0)),
            pl.BlockSpec((1, D), lambda i: (0, 0)),
            pl.BlockSpec((NPAIRS, 2 * D), lambda i: (0, 0)),
            pl.BlockSpec((2, D), lambda i: (0, 0)),
            pl.BlockSpec((1, D), lambda i: (0, 0)),
            pl.BlockSpec((1, D), lambda i: (0, 0)),
        ],
        out_specs=pl.BlockSpec((NB, NPAIRS, 2 * D), lambda i: (i, 0, 0)),
        out_shape=jax.ShapeDtypeStruct((B, NPAIRS, 2 * D), jnp.float32),
        interpret=interpret,
    )(staging_pairs, staging_pairs,
      cev[:, :, 0], cev[:, :, 1], cev[:, :, 2],
      cod[:, :, 0], cod[:, :, 1], cod[:, :, 2],
      W_path, b_path.reshape(1, D), pos_table.reshape(NPAIRS, 2 * D),
      type_table, gamma.reshape(1, D), beta.reshape(1, D))
    return out.reshape(B, SEQ, D)


def kernel(path_coords, char_tokens, cls_token, sep_token, W_path, b_path,
           char_table, pos_table, type_table, gamma, beta):
    idx = _build_idx(char_tokens, cls_token, sep_token).reshape(
        NPAD // GROWS, GROWS)
    staging = _sc_gather()(idx, char_table)
    staging_pairs = staging.reshape(NPAD // 2, 2 * D)
    return _tc_call(staging_pairs, path_coords, W_path, b_path, pos_table,
                    type_table, gamma, beta)


# speedup vs baseline: 1.3319x; 1.0276x over previous
import functools

import jax
import jax.numpy as jnp
from jax import lax
from jax.experimental import pallas as pl
from jax.experimental.pallas import tpu as pltpu
from jax.experimental.pallas import tpu_sc as plsc

B = 4096
PATH_LEN = 50
CHAR_LEN = 200
D = 64
SEQ = 1 + PATH_LEN + 1 + CHAR_LEN
EPS = 1e-5

NC, NS = 2, 16
NW = NC * NS
GROWS = 128
GPC = 4
CHUNK = GROWS * GPC
N_REAL = B * (CHAR_LEN + 2)
CPW = -(-N_REAL // (NW * CHUNK))
NPAD = NW * CPW * CHUNK
CPW_FAST = 56
CPW_SLOW = 2 * CPW - CPW_FAST
CS_PAIR0 = B * CHAR_LEN // 2


def _sc_gather_body(idx_hbm, table_hbm, out_hbm, idx_v, rows_v,
                    sidx0, sidx1, sg0, sg1, so0, so1):
    core = lax.axis_index("c")
    sub = lax.axis_index("s")
    base_chunk = jnp.where(core == 0, sub * CPW_FAST,
                           NS * CPW_FAST + sub * CPW_SLOW)
    count = jnp.where(core == 0, CPW_FAST, CPW_SLOW)
    sidx = (sidx0, sidx1)
    sg = (sg0, sg1)
    so = (so0, so1)

    def idx_start(c, b):
        pltpu.async_copy(idx_hbm.at[pl.ds((base_chunk + c) * GPC, GPC)],
                         idx_v.at[b], sidx[b])

    idx_start(0, 0)
    idx_start(1, 1)

    def it_body(it, carry):
        for b in range(2):
            c = 2 * it + b

            @pl.when(it > 0)
            def _():
                pltpu.make_async_copy(
                    rows_v.at[b], out_hbm.at[pl.ds(0, CHUNK)], so[b]).wait()

            pltpu.make_async_copy(
                idx_hbm.at[pl.ds(0, GPC)], idx_v.at[b], sidx[b]).wait()

            gathers = [
                pltpu.async_copy(
                    table_hbm.at[idx_v.at[b].at[j]],
                    rows_v.at[b].at[pl.ds(j * GROWS, GROWS)],
                    sg[b],
                )
                for j in range(GPC)
            ]
            for cp in gathers:
                cp.wait()

            @pl.when(c + 2 < count)
            def _():
                idx_start(c + 2, b)

            pltpu.async_copy(
                rows_v.at[b],
                out_hbm.at[pl.ds((base_chunk + c) * CHUNK, CHUNK)],
                so[b])
        return carry

    lax.fori_loop(0, count // 2, it_body, 0)
    for b in range(2):
        pltpu.make_async_copy(
            rows_v.at[b], out_hbm.at[pl.ds(0, CHUNK)], so[b]).wait()


@functools.lru_cache(maxsize=1)
def _sc_gather():
    return pl.kernel(
        _sc_gather_body,
        mesh=plsc.VectorSubcoreMesh(core_axis_name="c", subcore_axis_name="s"),
        out_type=jax.ShapeDtypeStruct((NPAD, D), jnp.float32),
        scratch_types=[
            pltpu.VMEM((2, GPC, GROWS), jnp.int32),
            pltpu.VMEM((2, CHUNK, D), jnp.float32),
            pltpu.SemaphoreType.DMA,
            pltpu.SemaphoreType.DMA,
            pltpu.SemaphoreType.DMA,
            pltpu.SemaphoreType.DMA,
            pltpu.SemaphoreType.DMA,
            pltpu.SemaphoreType.DMA,
        ],
        compiler_params=pltpu.CompilerParams(use_tc_tiling_on_sc=False),
    )


NB = 64
NPAIR_LO = 26
NPAIR_HI = 100
NPAIRS = NPAIR_LO + NPAIR_HI


def _tc_finish(char_ref, cs_ref, cevx_ref, cevy_ref, cevz_ref,
               codx_ref, cody_ref, codz_ref, w_ref, b_ref,
               pospair_ref, type_ref, gamma_ref, beta_ref, out_ref):
    g2 = jnp.concatenate([gamma_ref[...], gamma_ref[...]], axis=-1)
    b2 = jnp.concatenate([beta_ref[...], beta_ref[...]], axis=-1)
    t0 = type_ref[0:1, :]
    t00 = jnp.concatenate([t0, t0], axis=-1)[None]
    t1 = type_ref[1:2, :]
    t11 = jnp.concatenate([t1, t1], axis=-1)[None]

    ri = lax.broadcasted_iota(jnp.int32, (128, 128), 0) // 64
    ci = lax.broadcasted_iota(jnp.int32, (128, 128), 1) // 64
    m_half = (ri == ci).astype(jnp.float32)

    def ln2(x):
        r, p, _ = x.shape
        x2 = x.reshape(r * p, 128)
        s = lax.dot(x2, m_half)
        q = lax.dot(x2 * x2, m_half)
        mu = s * (1.0 / 64.0)
        v = q * (1.0 / 64.0) - mu * mu
        y = (x2 - mu) * lax.rsqrt(v + EPS) * g2 + b2
        return y.reshape(r, p, 128)

    w0 = w_ref[0:1, :]
    w1 = w_ref[1:2, :]
    w2 = w_ref[2:3, :]
    bp = b_ref[...]
    pe_l = (cevx_ref[...][:, :, None] * w0 + cevy_ref[...][:, :, None] * w1
            + cevz_ref[...][:, :, None] * w2 + bp)
    pe_r = (codx_ref[...][:, :, None] * w0 + cody_ref[...][:, :, None] * w1
            + codz_ref[...][:, :, None] * w2 + bp)
    lo = jnp.concatenate([pe_l, pe_r], axis=-1)
    pi = lax.broadcasted_iota(jnp.int32, (1, NPAIR_LO, 128), 1)
    li = lax.broadcasted_iota(jnp.int32, (1, NPAIR_LO, 128), 2)
    cs_mask = ((pi == 0) & (li < D)) | ((pi == NPAIR_LO - 1) & (li >= D))
    lo = jnp.where(cs_mask, cs_ref[...][:, None, :], lo)
    x_lo = lo + pospair_ref[0:NPAIR_LO, :][None] + t00
    out_ref[:, 0:NPAIR_LO, :] = ln2(x_lo)

    x_hi = (char_ref[...].reshape(NB, NPAIR_HI, 2 * D)
            + pospair_ref[NPAIR_LO:NPAIRS, :][None] + t11)
    out_ref[:, NPAIR_LO:NPAIRS, :] = ln2(x_hi)


def _build_idx(char_tokens, cls_token, sep_token):
    return jnp.concatenate([
        char_tokens.reshape(-1).astype(jnp.int32),
        jnp.concatenate([cls_token.astype(jnp.int32),
                         sep_token.astype(jnp.int32)], axis=1).reshape(-1),
        jnp.zeros((NPAD - N_REAL,), jnp.int32),
    ])


def _tc_call(staging_pairs, path_coords, W_path, b_path, pos_table,
             type_table, gamma, beta, interpret=False):

    zero3 = jnp.zeros((B, 1, 3), path_coords.dtype)
    cext = jnp.concatenate([zero3, path_coords, zero3], axis=1)
    cev = cext[:, 0::2, :]
    cod = cext[:, 1::2, :]

    grid = (B // NB,)
    cs_block0 = CS_PAIR0 // NB
    out = pl.pallas_call(
        _tc_finish,
        grid=grid,
        in_specs=[
            pl.BlockSpec((NB * NPAIR_HI, 2 * D), lambda i: (i, 0)),
            pl.BlockSpec((NB, 2 * D), lambda i: (cs_block0 + i, 0)),
            pl.BlockSpec((NB, NPAIR_LO), lambda i: (i, 0)),
            pl.BlockSpec((NB, NPAIR_LO), lambda i: (i, 0)),
            pl.BlockSpec((NB, NPAIR_LO), lambda i: (i, 0)),
            pl.BlockSpec((NB, NPAIR_LO), lambda i: (i, 0)),
            pl.BlockSpec((NB, NPAIR_LO), lambda i: (i, 0)),
            pl.BlockSpec((NB, NPAIR_LO), lambda i: (i, 0)),
            pl.BlockSpec((3, D), lambda i: (0, 0)),
            pl.BlockSpec((1, D), lambda i: (0, 0)),
            pl.BlockSpec((NPAIRS, 2 * D), lambda i: (0, 0)),
            pl.BlockSpec((2, D), lambda i: (0, 0)),
            pl.BlockSpec((1, D), lambda i: (0, 0)),
            pl.BlockSpec((1, D), lambda i: (0, 0)),
        ],
        out_specs=pl.BlockSpec((NB, NPAIRS, 2 * D), lambda i: (i, 0, 0)),
        out_shape=jax.ShapeDtypeStruct((B, NPAIRS, 2 * D), jnp.float32),
        interpret=interpret,
    )(staging_pairs, staging_pairs,
      cev[:, :, 0], cev[:, :, 1], cev[:, :, 2],
      cod[:, :, 0], cod[:, :, 1], cod[:, :, 2],
      W_path, b_path.reshape(1, D), pos_table.reshape(NPAIRS, 2 * D),
      type_table, gamma.reshape(1, D), beta.reshape(1, D))
    return out.reshape(B, SEQ, D)


def kernel(path_coords, char_tokens, cls_token, sep_token, W_path, b_path,
           char_table, pos_table, type_table, gamma, beta):
    idx = _build_idx(char_tokens, cls_token, sep_token).reshape(
        NPAD // GROWS, GROWS)
    staging = _sc_gather()(idx, char_table)
    staging_pairs = staging.reshape(NPAD // 2, 2 * D)
    return _tc_call(staging_pairs, path_coords, W_path, b_path, pos_table,
                    type_table, gamma, beta)


# speedup vs baseline: 1.3391x; 1.0054x over previous
import functools

import jax
import jax.numpy as jnp
from jax import lax
from jax.experimental import pallas as pl
from jax.experimental.pallas import tpu as pltpu
from jax.experimental.pallas import tpu_sc as plsc

B = 4096
PATH_LEN = 50
CHAR_LEN = 200
D = 64
SEQ = 1 + PATH_LEN + 1 + CHAR_LEN
EPS = 1e-5

NC, NS = 2, 16
NW = NC * NS
GROWS = 128
GPC = 4
CHUNK = GROWS * GPC
N_REAL = B * (CHAR_LEN + 2)
CPW = -(-N_REAL // (NW * CHUNK))
NPAD = NW * CPW * CHUNK
CPW_FAST = 48
CPW_SLOW = 2 * CPW - CPW_FAST
CS_PAIR0 = B * CHAR_LEN // 2


def _sc_gather_body(idx_hbm, table_hbm, out_hbm, idx_v, rows_v,
                    sidx0, sidx1, sg0, sg1, so0, so1):
    core = lax.axis_index("c")
    sub = lax.axis_index("s")
    base_chunk = jnp.where(core == 0, sub * CPW_FAST,
                           NS * CPW_FAST + sub * CPW_SLOW)
    count = jnp.where(core == 0, CPW_FAST, CPW_SLOW)
    sidx = (sidx0, sidx1)
    sg = (sg0, sg1)
    so = (so0, so1)

    def idx_start(c, b):
        pltpu.async_copy(idx_hbm.at[pl.ds((base_chunk + c) * GPC, GPC)],
                         idx_v.at[b], sidx[b])

    idx_start(0, 0)
    idx_start(1, 1)

    def it_body(it, carry):
        for b in range(2):
            c = 2 * it + b

            @pl.when(it > 0)
            def _():
                pltpu.make_async_copy(
                    rows_v.at[b], out_hbm.at[pl.ds(0, CHUNK)], so[b]).wait()

            pltpu.make_async_copy(
                idx_hbm.at[pl.ds(0, GPC)], idx_v.at[b], sidx[b]).wait()

            gathers = [
                pltpu.async_copy(
                    table_hbm.at[idx_v.at[b].at[j]],
                    rows_v.at[b].at[pl.ds(j * GROWS, GROWS)],
                    sg[b],
                )
                for j in range(GPC)
            ]
            for cp in gathers:
                cp.wait()

            @pl.when(c + 2 < count)
            def _():
                idx_start(c + 2, b)

            pltpu.async_copy(
                rows_v.at[b],
                out_hbm.at[pl.ds((base_chunk + c) * CHUNK, CHUNK)],
                so[b])
        return carry

    lax.fori_loop(0, count // 2, it_body, 0)
    for b in range(2):
        pltpu.make_async_copy(
            rows_v.at[b], out_hbm.at[pl.ds(0, CHUNK)], so[b]).wait()


@functools.lru_cache(maxsize=1)
def _sc_gather():
    return pl.kernel(
        _sc_gather_body,
        mesh=plsc.VectorSubcoreMesh(core_axis_name="c", subcore_axis_name="s"),
        out_type=jax.ShapeDtypeStruct((NPAD, D), jnp.float32),
        scratch_types=[
            pltpu.VMEM((2, GPC, GROWS), jnp.int32),
            pltpu.VMEM((2, CHUNK, D), jnp.float32),
            pltpu.SemaphoreType.DMA,
            pltpu.SemaphoreType.DMA,
            pltpu.SemaphoreType.DMA,
            pltpu.SemaphoreType.DMA,
            pltpu.SemaphoreType.DMA,
            pltpu.SemaphoreType.DMA,
        ],
        compiler_params=pltpu.CompilerParams(use_tc_tiling_on_sc=False),
    )


NB = 64
NPAIR_LO = 26
NPAIR_HI = 100
NPAIRS = NPAIR_LO + NPAIR_HI


def _tc_finish(char_ref, cs_ref, cevx_ref, cevy_ref, cevz_ref,
               codx_ref, cody_ref, codz_ref, w_ref, b_ref,
               pospair_ref, type_ref, gamma_ref, beta_ref, out_ref):
    g2 = jnp.concatenate([gamma_ref[...], gamma_ref[...]], axis=-1)
    b2 = jnp.concatenate([beta_ref[...], beta_ref[...]], axis=-1)
    t0 = type_ref[0:1, :]
    t00 = jnp.concatenate([t0, t0], axis=-1)[None]
    t1 = type_ref[1:2, :]
    t11 = jnp.concatenate([t1, t1], axis=-1)[None]

    ri = lax.broadcasted_iota(jnp.int32, (128, 128), 0) // 64
    ci = lax.broadcasted_iota(jnp.int32, (128, 128), 1) // 64
    m_half = (ri == ci).astype(jnp.float32)

    def ln2(x):
        r, p, _ = x.shape
        x2 = x.reshape(r * p, 128)
        s = lax.dot(x2, m_half)
        q = lax.dot(x2 * x2, m_half)
        mu = s * (1.0 / 64.0)
        v = q * (1.0 / 64.0) - mu * mu
        y = (x2 - mu) * lax.rsqrt(v + EPS) * g2 + b2
        return y.reshape(r, p, 128)

    w0 = w_ref[0:1, :]
    w1 = w_ref[1:2, :]
    w2 = w_ref[2:3, :]
    bp = b_ref[...]
    pe_l = (cevx_ref[...][:, :, None] * w0 + cevy_ref[...][:, :, None] * w1
            + cevz_ref[...][:, :, None] * w2 + bp)
    pe_r = (codx_ref[...][:, :, None] * w0 + cody_ref[...][:, :, None] * w1
            + codz_ref[...][:, :, None] * w2 + bp)
    lo = jnp.concatenate([pe_l, pe_r], axis=-1)
    pi = lax.broadcasted_iota(jnp.int32, (1, NPAIR_LO, 128), 1)
    li = lax.broadcasted_iota(jnp.int32, (1, NPAIR_LO, 128), 2)
    cs_mask = ((pi == 0) & (li < D)) | ((pi == NPAIR_LO - 1) & (li >= D))
    lo = jnp.where(cs_mask, cs_ref[...][:, None, :], lo)
    x_lo = lo + pospair_ref[0:NPAIR_LO, :][None] + t00
    out_ref[:, 0:NPAIR_LO, :] = ln2(x_lo)

    x_hi = (char_ref[...].reshape(NB, NPAIR_HI, 2 * D)
            + pospair_ref[NPAIR_LO:NPAIRS, :][None] + t11)
    out_ref[:, NPAIR_LO:NPAIRS, :] = ln2(x_hi)


def _build_idx(char_tokens, cls_token, sep_token):
    return jnp.concatenate([
        char_tokens.reshape(-1).astype(jnp.int32),
        jnp.concatenate([cls_token.astype(jnp.int32),
                         sep_token.astype(jnp.int32)], axis=1).reshape(-1),
        jnp.zeros((NPAD - N_REAL,), jnp.int32),
    ])


def _tc_call(staging_pairs, path_coords, W_path, b_path, pos_table,
             type_table, gamma, beta, interpret=False):

    zero3 = jnp.zeros((B, 1, 3), path_coords.dtype)
    cext = jnp.concatenate([zero3, path_coords, zero3], axis=1)
    cev = cext[:, 0::2, :]
    cod = cext[:, 1::2, :]

    grid = (B // NB,)
    cs_block0 = CS_PAIR0 // NB
    out = pl.pallas_call(
        _tc_finish,
        grid=grid,
        in_specs=[
            pl.BlockSpec((NB * NPAIR_HI, 2 * D), lambda i: (i, 0)),
            pl.BlockSpec((NB, 2 * D), lambda i: (cs_block0 + i, 0)),
            pl.BlockSpec((NB, NPAIR_LO), lambda i: (i, 0)),
            pl.BlockSpec((NB, NPAIR_LO), lambda i: (i, 0)),
            pl.BlockSpec((NB, NPAIR_LO), lambda i: (i, 0)),
            pl.BlockSpec((NB, NPAIR_LO), lambda i: (i, 0)),
            pl.BlockSpec((NB, NPAIR_LO), lambda i: (i, 0)),
            pl.BlockSpec((NB, NPAIR_LO), lambda i: (i, 0)),
            pl.BlockSpec((3, D), lambda i: (0, 0)),
            pl.BlockSpec((1, D), lambda i: (0, 0)),
            pl.BlockSpec((NPAIRS, 2 * D), lambda i: (0, 0)),
            pl.BlockSpec((2, D), lambda i: (0, 0)),
            pl.BlockSpec((1, D), lambda i: (0, 0)),
            pl.BlockSpec((1, D), lambda i: (0, 0)),
        ],
        out_specs=pl.BlockSpec((NB, NPAIRS, 2 * D), lambda i: (i, 0, 0)),
        out_shape=jax.ShapeDtypeStruct((B, NPAIRS, 2 * D), jnp.float32),
        interpret=interpret,
    )(staging_pairs, staging_pairs,
      cev[:, :, 0], cev[:, :, 1], cev[:, :, 2],
      cod[:, :, 0], cod[:, :, 1], cod[:, :, 2],
      W_path, b_path.reshape(1, D), pos_table.reshape(NPAIRS, 2 * D),
      type_table, gamma.reshape(1, D), beta.reshape(1, D))
    return out.reshape(B, SEQ, D)


def kernel(path_coords, char_tokens, cls_token, sep_token, W_path, b_path,
           char_table, pos_table, type_table, gamma, beta):
    idx = _build_idx(char_tokens, cls_token, sep_token).reshape(
        NPAD // GROWS, GROWS)
    staging = _sc_gather()(idx, char_table)
    staging_pairs = staging.reshape(NPAD // 2, 2 * D)
    return _tc_call(staging_pairs, path_coords, W_path, b_path, pos_table,
                    type_table, gamma, beta)


# speedup vs baseline: 1.3434x; 1.0032x over previous
import functools

import jax
import jax.numpy as jnp
from jax import lax
from jax.experimental import pallas as pl
from jax.experimental.pallas import tpu as pltpu
from jax.experimental.pallas import tpu_sc as plsc

B = 4096
PATH_LEN = 50
CHAR_LEN = 200
D = 64
SEQ = 1 + PATH_LEN + 1 + CHAR_LEN
EPS = 1e-5

NC, NS = 2, 16
NW = NC * NS
GROWS = 128
GPC = 4
CHUNK = GROWS * GPC
N_REAL = B * (CHAR_LEN + 2)
CPW = -(-N_REAL // (NW * CHUNK))
NPAD = NW * CPW * CHUNK
CPW_FAST = 52
CPW_SLOW = 2 * CPW - CPW_FAST
CS_PAIR0 = B * CHAR_LEN // 2


def _sc_gather_body(idx_hbm, table_hbm, out_hbm, idx_v, rows_v,
                    sidx0, sidx1, sg0, sg1, so0, so1):
    core = lax.axis_index("c")
    sub = lax.axis_index("s")
    base_chunk = jnp.where(core == 0, sub * CPW_FAST,
                           NS * CPW_FAST + sub * CPW_SLOW)
    count = jnp.where(core == 0, CPW_FAST, CPW_SLOW)
    sidx = (sidx0, sidx1)
    sg = (sg0, sg1)
    so = (so0, so1)

    def idx_start(c, b):
        pltpu.async_copy(idx_hbm.at[pl.ds((base_chunk + c) * GPC, GPC)],
                         idx_v.at[b], sidx[b])

    idx_start(0, 0)
    idx_start(1, 1)

    def it_body(it, carry):
        for b in range(2):
            c = 2 * it + b

            @pl.when(it > 0)
            def _():
                pltpu.make_async_copy(
                    rows_v.at[b], out_hbm.at[pl.ds(0, CHUNK)], so[b]).wait()

            pltpu.make_async_copy(
                idx_hbm.at[pl.ds(0, GPC)], idx_v.at[b], sidx[b]).wait()

            gathers = [
                pltpu.async_copy(
                    table_hbm.at[idx_v.at[b].at[j]],
                    rows_v.at[b].at[pl.ds(j * GROWS, GROWS)],
                    sg[b],
                )
                for j in range(GPC)
            ]
            for cp in gathers:
                cp.wait()

            @pl.when(c + 2 < count)
            def _():
                idx_start(c + 2, b)

            pltpu.async_copy(
                rows_v.at[b],
                out_hbm.at[pl.ds((base_chunk + c) * CHUNK, CHUNK)],
                so[b])
        return carry

    lax.fori_loop(0, count // 2, it_body, 0)
    for b in range(2):
        pltpu.make_async_copy(
            rows_v.at[b], out_hbm.at[pl.ds(0, CHUNK)], so[b]).wait()


@functools.lru_cache(maxsize=1)
def _sc_gather():
    return pl.kernel(
        _sc_gather_body,
        mesh=plsc.VectorSubcoreMesh(core_axis_name="c", subcore_axis_name="s"),
        out_type=jax.ShapeDtypeStruct((NPAD, D), jnp.float32),
        scratch_types=[
            pltpu.VMEM((2, GPC, GROWS), jnp.int32),
            pltpu.VMEM((2, CHUNK, D), jnp.float32),
            pltpu.SemaphoreType.DMA,
            pltpu.SemaphoreType.DMA,
            pltpu.SemaphoreType.DMA,
            pltpu.SemaphoreType.DMA,
            pltpu.SemaphoreType.DMA,
            pltpu.SemaphoreType.DMA,
        ],
        compiler_params=pltpu.CompilerParams(use_tc_tiling_on_sc=False),
    )


NB = 64
NPAIR_LO = 26
NPAIR_HI = 100
NPAIRS = NPAIR_LO + NPAIR_HI


def _tc_finish(char_ref, cs_ref, cevx_ref, cevy_ref, cevz_ref,
               codx_ref, cody_ref, codz_ref, w_ref, b_ref,
               pospair_ref, type_ref, gamma_ref, beta_ref, out_ref):
    g2 = jnp.concatenate([gamma_ref[...], gamma_ref[...]], axis=-1)
    b2 = jnp.concatenate([beta_ref[...], beta_ref[...]], axis=-1)
    t0 = type_ref[0:1, :]
    t00 = jnp.concatenate([t0, t0], axis=-1)[None]
    t1 = type_ref[1:2, :]
    t11 = jnp.concatenate([t1, t1], axis=-1)[None]

    ri = lax.broadcasted_iota(jnp.int32, (128, 128), 0) // 64
    ci = lax.broadcasted_iota(jnp.int32, (128, 128), 1) // 64
    m_half = (ri == ci).astype(jnp.float32)

    def ln2(x):
        r, p, _ = x.shape
        x2 = x.reshape(r * p, 128)
        s = lax.dot(x2, m_half)
        q = lax.dot(x2 * x2, m_half)
        mu = s * (1.0 / 64.0)
        v = q * (1.0 / 64.0) - mu * mu
        y = (x2 - mu) * lax.rsqrt(v + EPS) * g2 + b2
        return y.reshape(r, p, 128)

    w0 = w_ref[0:1, :]
    w1 = w_ref[1:2, :]
    w2 = w_ref[2:3, :]
    bp = b_ref[...]
    pe_l = (cevx_ref[...][:, :, None] * w0 + cevy_ref[...][:, :, None] * w1
            + cevz_ref[...][:, :, None] * w2 + bp)
    pe_r = (codx_ref[...][:, :, None] * w0 + cody_ref[...][:, :, None] * w1
            + codz_ref[...][:, :, None] * w2 + bp)
    lo = jnp.concatenate([pe_l, pe_r], axis=-1)
    pi = lax.broadcasted_iota(jnp.int32, (1, NPAIR_LO, 128), 1)
    li = lax.broadcasted_iota(jnp.int32, (1, NPAIR_LO, 128), 2)
    cs_mask = ((pi == 0) & (li < D)) | ((pi == NPAIR_LO - 1) & (li >= D))
    lo = jnp.where(cs_mask, cs_ref[...][:, None, :], lo)
    x_lo = lo + pospair_ref[0:NPAIR_LO, :][None] + t00
    out_ref[:, 0:NPAIR_LO, :] = ln2(x_lo)

    x_hi = (char_ref[...].reshape(NB, NPAIR_HI, 2 * D)
            + pospair_ref[NPAIR_LO:NPAIRS, :][None] + t11)
    out_ref[:, NPAIR_LO:NPAIRS, :] = ln2(x_hi)


def _build_idx(char_tokens, cls_token, sep_token):
    return jnp.concatenate([
        char_tokens.reshape(-1).astype(jnp.int32),
        jnp.concatenate([cls_token.astype(jnp.int32),
                         sep_token.astype(jnp.int32)], axis=1).reshape(-1),
        jnp.zeros((NPAD - N_REAL,), jnp.int32),
    ])


def _tc_call(staging_pairs, path_coords, W_path, b_path, pos_table,
             type_table, gamma, beta, interpret=False):

    zero3 = jnp.zeros((B, 1, 3), path_coords.dtype)
    cext = jnp.concatenate([zero3, path_coords, zero3], axis=1)
    cev = cext[:, 0::2, :]
    cod = cext[:, 1::2, :]

    grid = (B // NB,)
    cs_block0 = CS_PAIR0 // NB
    out = pl.pallas_call(
        _tc_finish,
        grid=grid,
        in_specs=[
            pl.BlockSpec((NB * NPAIR_HI, 2 * D), lambda i: (i, 0)),
            pl.BlockSpec((NB, 2 * D), lambda i: (cs_block0 + i, 0)),
            pl.BlockSpec((NB, NPAIR_LO), lambda i: (i, 0)),
            pl.BlockSpec((NB, NPAIR_LO), lambda i: (i, 0)),
            pl.BlockSpec((NB, NPAIR_LO), lambda i: (i, 0)),
            pl.BlockSpec((NB, NPAIR_LO), lambda i: (i, 0)),
            pl.BlockSpec((NB, NPAIR_LO), lambda i: (i, 0)),
            pl.BlockSpec((NB, NPAIR_LO), lambda i: (i, 0)),
            pl.BlockSpec((3, D), lambda i: (0, 0)),
            pl.BlockSpec((1, D), lambda i: (0, 0)),
            pl.BlockSpec((NPAIRS, 2 * D), lambda i: (0, 0)),
            pl.BlockSpec((2, D), lambda i: (0, 0)),
            pl.BlockSpec((1, D), lambda i: (0, 0)),
            pl.BlockSpec((1, D), lambda i: (0, 0)),
        ],
        out_specs=pl.BlockSpec((NB, NPAIRS, 2 * D), lambda i: (i, 0, 0)),
        out_shape=jax.ShapeDtypeStruct((B, NPAIRS, 2 * D), jnp.float32),
        interpret=interpret,
    )(staging_pairs, staging_pairs,
      cev[:, :, 0], cev[:, :, 1], cev[:, :, 2],
      cod[:, :, 0], cod[:, :, 1], cod[:, :, 2],
      W_path, b_path.reshape(1, D), pos_table.reshape(NPAIRS, 2 * D),
      type_table, gamma.reshape(1, D), beta.reshape(1, D))
    return out.reshape(B, SEQ, D)


def kernel(path_coords, char_tokens, cls_token, sep_token, W_path, b_path,
           char_table, pos_table, type_table, gamma, beta):
    idx = _build_idx(char_tokens, cls_token, sep_token).reshape(
        NPAD // GROWS, GROWS)
    staging = _sc_gather()(idx, char_table)
    staging_pairs = staging.reshape(NPAD // 2, 2 * D)
    return _tc_call(staging_pairs, path_coords, W_path, b_path, pos_table,
                    type_table, gamma, beta)
